# Initial kernel scaffold; baseline (speedup 1.0000x reference)
import jax
import jax.numpy as jnp
from jax.experimental import pallas as pl


def kernel(x):
    raise NotImplementedError("write your pallas kernel here")


# speedup vs baseline: 10.3992x; 10.3992x over previous
import jax
import jax.numpy as jnp
from jax import lax
from jax.experimental import pallas as pl

_K = 512
_ROWS_PER_BLOCK = 8


def _topk_mask_body(x_ref, o_ref):
    x = x_ref[...]
    br, n = x.shape
    xi = lax.bitcast_convert_type(x, jnp.int32)
    key = xi ^ ((xi >> 31) & jnp.int32(0x7FFFFFFF))

    k = jnp.int32(_K)

    def count_ge(cand):
        return jnp.sum((key >= cand).astype(jnp.int32), axis=1, keepdims=True)

    cnt_pos = count_ge(jnp.zeros((br, 1), jnp.int32))
    prefix0 = jnp.where(cnt_pos >= k, jnp.int32(0), jnp.int32(-(2**31)))

    def bit_body(i, prefix):
        cand = prefix | (jnp.int32(1) << (jnp.int32(30) - i))
        cnt = count_ge(cand)
        return jnp.where(cnt >= k, cand, prefix)

    t = lax.fori_loop(0, 31, bit_body, prefix0)

    gt = key > t
    eq = key == t
    cnt_gt = jnp.sum(gt.astype(jnp.int32), axis=1, keepdims=True)
    cnt_eq = jnp.sum(eq.astype(jnp.int32), axis=1, keepdims=True)
    r = k - cnt_gt

    idx = lax.broadcasted_iota(jnp.int32, (br, n), 1)
    need_tie_search = jnp.any(r < cnt_eq)

    def tie_search(_):
        def bodyI(i, c):
            cand = c | (jnp.int32(1) << (jnp.int32(15) - i))
            g = jnp.sum((eq & (idx < cand)).astype(jnp.int32),
                        axis=1, keepdims=True)
            return jnp.where(g < r, cand, c)

        return lax.fori_loop(0, 16, bodyI, jnp.zeros((br, 1), jnp.int32))

    i_star = lax.cond(need_tie_search, tie_search,
                      lambda _: jnp.full((br, 1), n, jnp.int32), None)

    keep = gt | (eq & (idx <= i_star))
    o_ref[...] = jnp.where(keep, x, jnp.float32(0.0))


def kernel(x):
    b, n = x.shape
    grid = b // _ROWS_PER_BLOCK
    return pl.pallas_call(
        _topk_mask_body,
        grid=(grid,),
        in_specs=[pl.BlockSpec((_ROWS_PER_BLOCK, n), lambda i: (i, 0))],
        out_specs=pl.BlockSpec((_ROWS_PER_BLOCK, n), lambda i: (i, 0)),
        out_shape=jax.ShapeDtypeStruct((b, n), jnp.float32),
    )(x)
